# Initial kernel scaffold; baseline (speedup 1.0000x reference)
import jax
import jax.numpy as jnp
from jax.experimental import pallas as pl


def kernel(x, w_fused, b_fused):
    raise NotImplementedError("write your pallas kernel here")


# speedup vs baseline: 1.0404x; 1.0404x over previous
import jax
import jax.numpy as jnp
from jax.experimental import pallas as pl
from jax.experimental.pallas import tpu as pltpu

_LANES = 128
_SUBLANES = 8


def _round_up(x, m):
    return ((x + m - 1) // m) * m


def _fused_affine_kernel(x_ref, w_ref, b_ref, o_ref):
    xb = x_ref[...].astype(jnp.bfloat16)
    y = jnp.dot(xb, w_ref[...], preferred_element_type=jnp.float32)
    o_ref[...] = y + b_ref[...]


def kernel(x, w_fused, b_fused):
    n, in_f = x.shape
    out_f = w_fused.shape[1]

    in_pad = _round_up(in_f, _LANES)
    out_pad = _round_up(out_f, _LANES)
    w_p = w_fused
    b_p = b_fused
    if in_pad != in_f or out_pad != out_f:
        w_p = jnp.zeros((in_pad, out_pad), jnp.float32).at[:in_f, :out_f].set(w_fused)
        b_p = jnp.zeros((1, out_pad), jnp.float32).at[:, :out_f].set(b_fused)
    w_bf = w_p.astype(jnp.bfloat16)

    x_p = x
    if in_pad != in_f:
        x_p = jnp.zeros((n, in_pad), jnp.float32).at[:, :in_f].set(x)

    tm = min(1024, _round_up(n, _SUBLANES))
    n_pad = _round_up(n, tm)
    if n_pad != n:
        x_p = jnp.zeros((n_pad, in_pad), x_p.dtype).at[:n, :].set(x_p)

    grid = (n_pad // tm,)
    y_pad = pl.pallas_call(
        _fused_affine_kernel,
        out_shape=jax.ShapeDtypeStruct((n_pad, out_pad), jnp.float32),
        grid=grid,
        in_specs=[
            pl.BlockSpec((tm, in_pad), lambda i: (i, 0)),
            pl.BlockSpec((in_pad, out_pad), lambda i: (0, 0)),
            pl.BlockSpec((1, out_pad), lambda i: (0, 0)),
        ],
        out_specs=pl.BlockSpec((tm, out_pad), lambda i: (i, 0)),
        compiler_params=pltpu.CompilerParams(
            dimension_semantics=("parallel",)),
        cost_estimate=pl.CostEstimate(
            flops=2 * n_pad * in_pad * out_pad, transcendentals=0,
            bytes_accessed=4 * (n_pad * in_pad + n_pad * out_pad)
            + 2 * in_pad * out_pad),
    )(x_p, w_bf, b_p)

    if n_pad != n or out_pad != out_f:
        return y_pad[:n, :out_f]
    return y_pad


# speedup vs baseline: 1.0425x; 1.0020x over previous
import jax
import jax.numpy as jnp
from jax.experimental import pallas as pl
from jax.experimental.pallas import tpu as pltpu

_LANES = 128
_SUBLANES = 8


def _round_up(x, m):
    return ((x + m - 1) // m) * m


def _fused_affine_kernel(x_ref, w_ref, b_ref, o_ref):
    xb = x_ref[...].astype(jnp.bfloat16)
    y = jnp.dot(xb, w_ref[...], preferred_element_type=jnp.float32)
    o_ref[...] = y + b_ref[...]


def kernel(x, w_fused, b_fused):
    n, in_f = x.shape
    out_f = w_fused.shape[1]

    in_pad = _round_up(in_f, _LANES)
    out_pad = _round_up(out_f, _LANES)
    w_p = w_fused
    b_p = b_fused
    if in_pad != in_f or out_pad != out_f:
        w_p = jnp.zeros((in_pad, out_pad), jnp.float32).at[:in_f, :out_f].set(w_fused)
        b_p = jnp.zeros((1, out_pad), jnp.float32).at[:, :out_f].set(b_fused)
    w_bf = w_p.astype(jnp.bfloat16)

    x_p = x
    if in_pad != in_f:
        x_p = jnp.zeros((n, in_pad), jnp.float32).at[:, :in_f].set(x)

    tm = min(2048, _round_up(n, _SUBLANES))
    n_pad = _round_up(n, tm)
    if n_pad != n:
        x_p = jnp.zeros((n_pad, in_pad), x_p.dtype).at[:n, :].set(x_p)

    grid = (n_pad // tm,)
    y_pad = pl.pallas_call(
        _fused_affine_kernel,
        out_shape=jax.ShapeDtypeStruct((n_pad, out_pad), jnp.float32),
        grid=grid,
        in_specs=[
            pl.BlockSpec((tm, in_pad), lambda i: (i, 0)),
            pl.BlockSpec((in_pad, out_pad), lambda i: (0, 0)),
            pl.BlockSpec((1, out_pad), lambda i: (0, 0)),
        ],
        out_specs=pl.BlockSpec((tm, out_pad), lambda i: (i, 0)),
        compiler_params=pltpu.CompilerParams(
            dimension_semantics=("parallel",)),
        cost_estimate=pl.CostEstimate(
            flops=2 * n_pad * in_pad * out_pad, transcendentals=0,
            bytes_accessed=4 * (n_pad * in_pad + n_pad * out_pad)
            + 2 * in_pad * out_pad),
    )(x_p, w_bf, b_p)

    if n_pad != n or out_pad != out_f:
        return y_pad[:n, :out_f]
    return y_pad


# speedup vs baseline: 1.0483x; 1.0056x over previous
import jax
import jax.numpy as jnp
from jax.experimental import pallas as pl
from jax.experimental.pallas import tpu as pltpu

_LANES = 128
_SUBLANES = 8


def _round_up(x, m):
    return ((x + m - 1) // m) * m


def _fused_affine_kernel(x0_ref, x1_ref, x2_ref, x3_ref, w_ref, b_ref, o_ref):
    w = w_ref[...]
    b = b_ref[...]
    tm2 = x0_ref.shape[0]
    for j, x_ref in enumerate((x0_ref, x1_ref, x2_ref, x3_ref)):
        xb = x_ref[...].astype(jnp.bfloat16)
        y = jnp.dot(xb, w, preferred_element_type=jnp.float32)
        o_ref[j * tm2:(j + 1) * tm2, :] = y + b


def kernel(x, w_fused, b_fused):
    n, in_f = x.shape
    out_f = w_fused.shape[1]

    in_pad = _round_up(in_f, _LANES)
    out_pad = _round_up(out_f, _LANES)
    w_p = w_fused
    b_p = b_fused
    if in_pad != in_f or out_pad != out_f:
        w_p = jnp.zeros((in_pad, out_pad), jnp.float32).at[:in_f, :out_f].set(w_fused)
        b_p = jnp.zeros((1, out_pad), jnp.float32).at[:, :out_f].set(b_fused)
    w_bf = w_p.astype(jnp.bfloat16)

    x_p = x
    if in_pad != in_f:
        x_p = jnp.zeros((n, in_pad), jnp.float32).at[:, :in_f].set(x)

    tm = min(2048, _round_up(n, 4 * _SUBLANES))
    n_pad = _round_up(n, tm)
    if n_pad != n:
        x_p = jnp.zeros((n_pad, in_pad), x_p.dtype).at[:n, :].set(x_p)
    tm2 = tm // 4

    grid = (n_pad // tm,)
    chunk = lambda j: pl.BlockSpec((tm2, in_pad), lambda i, j=j: (4 * i + j, 0))
    y_pad = pl.pallas_call(
        _fused_affine_kernel,
        out_shape=jax.ShapeDtypeStruct((n_pad, out_pad), jnp.float32),
        grid=grid,
        in_specs=[
            chunk(0), chunk(1), chunk(2), chunk(3),
            pl.BlockSpec((in_pad, out_pad), lambda i: (0, 0)),
            pl.BlockSpec((1, out_pad), lambda i: (0, 0)),
        ],
        out_specs=pl.BlockSpec((tm, out_pad), lambda i: (i, 0)),
        compiler_params=pltpu.CompilerParams(
            dimension_semantics=("parallel",)),
        cost_estimate=pl.CostEstimate(
            flops=2 * n_pad * in_pad * out_pad, transcendentals=0,
            bytes_accessed=4 * (n_pad * in_pad + n_pad * out_pad)
            + 2 * in_pad * out_pad),
    )(x_p, x_p, x_p, x_p, w_bf, b_p)

    if n_pad != n or out_pad != out_f:
        return y_pad[:n, :out_f]
    return y_pad


# speedup vs baseline: 1.1656x; 1.1118x over previous
import jax
import jax.numpy as jnp
from jax.experimental import pallas as pl
from jax.experimental.pallas import tpu as pltpu

_LANES = 128
_SUBLANES = 8


def _round_up(x, m):
    return ((x + m - 1) // m) * m


def _fused_affine_kernel(x0_ref, x1_ref, x2_ref, x3_ref, w_ref, b_ref, o_ref):
    w = w_ref[...]
    b = b_ref[...]
    tm2 = x0_ref.shape[0]
    for j, x_ref in enumerate((x0_ref, x1_ref, x2_ref, x3_ref)):
        y = jnp.dot(x_ref[...], w, preferred_element_type=jnp.float32)
        o_ref[j * tm2:(j + 1) * tm2, :] = y + b


def kernel(x, w_fused, b_fused):
    n, in_f = x.shape
    out_f = w_fused.shape[1]

    in_pad = _round_up(in_f, _LANES)
    out_pad = _round_up(out_f, _LANES)
    w_p = w_fused
    b_p = b_fused
    if in_pad != in_f or out_pad != out_f:
        w_p = jnp.zeros((in_pad, out_pad), jnp.float32).at[:in_f, :out_f].set(w_fused)
        b_p = jnp.zeros((1, out_pad), jnp.float32).at[:, :out_f].set(b_fused)

    x_p = x
    if in_pad != in_f:
        x_p = jnp.zeros((n, in_pad), jnp.float32).at[:, :in_f].set(x)

    tm = min(2048, _round_up(n, 4 * _SUBLANES))
    n_pad = _round_up(n, tm)
    if n_pad != n:
        x_p = jnp.zeros((n_pad, in_pad), x_p.dtype).at[:n, :].set(x_p)
    tm2 = tm // 4

    grid = (n_pad // tm,)
    chunk = lambda j: pl.BlockSpec((tm2, in_pad), lambda i, j=j: (4 * i + j, 0))
    y_pad = pl.pallas_call(
        _fused_affine_kernel,
        out_shape=jax.ShapeDtypeStruct((n_pad, out_pad), jnp.float32),
        grid=grid,
        in_specs=[
            chunk(0), chunk(1), chunk(2), chunk(3),
            pl.BlockSpec((in_pad, out_pad), lambda i: (0, 0)),
            pl.BlockSpec((1, out_pad), lambda i: (0, 0)),
        ],
        out_specs=pl.BlockSpec((tm, out_pad), lambda i: (i, 0)),
        compiler_params=pltpu.CompilerParams(
            dimension_semantics=("parallel",)),
        cost_estimate=pl.CostEstimate(
            flops=2 * n_pad * in_pad * out_pad, transcendentals=0,
            bytes_accessed=4 * (n_pad * in_pad + n_pad * out_pad
                                + in_pad * out_pad)),
    )(x_p, x_p, x_p, x_p, w_p, b_p)

    if n_pad != n or out_pad != out_f:
        return y_pad[:n, :out_f]
    return y_pad
